# baseline (speedup 1.0000x reference)
import dataclasses
import functools

import jax
import jax.numpy as jnp
from jax.experimental import pallas as pl
from jax.experimental.pallas import tpu as pltpu
from jax.experimental.pallas import tpu_sc as plsc

N = 50000
E = 1600000
R = 90

N_PAD = 50176
E_PAD = 1638400
ROWS_PER_TILE = E_PAD // 32 // 128
CHUNK_ROWS = 8
N_CHUNKS = ROWS_PER_TILE // CHUNK_ROWS
SLICE = N_PAD // 16


def _make_sc_agg(pairsum: bool):
    mesh = plsc.VectorSubcoreMesh(core_axis_name="c", subcore_axis_name="s")
    cp = pltpu.CompilerParams()
    for f, v in (("needs_layout_passes", False),
                 ("use_tc_tiling_on_sc", False)):
        if f in pltpu.CompilerParams.__dataclass_fields__:
            cp = dataclasses.replace(cp, **{f: v})

    @functools.partial(
        pl.kernel,
        compiler_params=cp,
        out_type=jax.ShapeDtypeStruct((2, N_PAD, 8), jnp.float32),
        mesh=mesh,
        scratch_types=[
            pltpu.VMEM((2, 1024), jnp.int32),
            pltpu.VMEM((2, 1024), jnp.int32),
            pltpu.VMEM((4, CHUNK_ROWS, 128), jnp.int32),
            pltpu.VMEM((2, 1024, 16), jnp.float32),
            pltpu.VMEM((2, 1024, 8), jnp.float32),
            pltpu.VMEM((R + 1, 8), jnp.float32),
            pltpu.VMEM_SHARED((N_PAD, 8), jnp.float32),
            pltpu.SemaphoreType.DMA,
            pltpu.SemaphoreType.DMA,
            pltpu.SemaphoreType.DMA,
            pltpu.SemaphoreType.DMA,
            pltpu.SemaphoreType.DMA,
            pltpu.SemaphoreType.DMA,
        ],
    )
    def sc_agg(x_hbm, src_hbm, et_hbm, dst_hbm, wf_hbm, z_hbm, out_hbm,
               src_buf, et_buf, dst_buf, xrow, msg, wf_buf, agg_sh,
               isem0, isem1, gsem0, gsem1, ssem0, ssem1):
        isems = (isem0, isem1)
        gsems = (gsem0, gsem1)
        ssems = (ssem0, ssem1)
        c = jax.lax.axis_index("c")
        s = jax.lax.axis_index("s")
        wid = c * 16 + s
        lanes = jax.lax.iota(jnp.int32, 16)
        cols = [jnp.full((16,), k, jnp.int32) for k in range(8)]
        zero16 = jnp.zeros((16,), jnp.float32)
        one16 = jnp.ones((16,), jnp.float32)
        base = wid * ROWS_PER_TILE

        pltpu.sync_copy(wf_hbm, wf_buf)
        pltpu.sync_copy(z_hbm.at[pl.ds(s * SLICE, SLICE)],
                        agg_sh.at[pl.ds(s * SLICE, SLICE)])

        @pl.loop(0, 64)
        def _init(g):
            rows = g * 16 + lanes
            for b in range(2):
                if pairsum:
                    for k in (3, 4, 5, 6, 7):
                        plsc.store_scatter(msg.at[b], [rows, cols[k]], zero16)
                else:
                    plsc.store_scatter(msg.at[b], [rows, cols[6]], one16)
                    plsc.store_scatter(msg.at[b], [rows, cols[7]], zero16)

        plsc.subcore_barrier()

        def load_idx(t, b, sem):
            r0 = base + t * CHUNK_ROWS
            e0 = r0 * 128
            h1 = pltpu.async_copy(src_hbm.at[pl.ds(e0, 1024)],
                                  src_buf.at[b], sem)
            h2 = pltpu.async_copy(et_hbm.at[pl.ds(e0, 1024)],
                                  et_buf.at[b], sem)
            h3 = pltpu.async_copy(dst_hbm.at[pl.ds(r0, CHUNK_ROWS)],
                                  dst_buf.at[t % 4], sem)
            return h1, h2, h3

        def wait_idx(b):
            pltpu.make_async_copy(src_hbm.at[pl.ds(0, 1024)],
                                  src_buf.at[b], isems[b]).wait()
            pltpu.make_async_copy(et_hbm.at[pl.ds(0, 1024)],
                                  et_buf.at[b], isems[b]).wait()
            pltpu.make_async_copy(dst_hbm.at[pl.ds(0, CHUNK_ROWS)],
                                  dst_buf.at[0], isems[b]).wait()

        def fire_gathers(b):
            for j in range(CHUNK_ROWS):
                pltpu.async_copy(
                    x_hbm.at[src_buf.at[b].at[pl.ds(j * 128, 128)]],
                    xrow.at[b].at[pl.ds(j * 128, 128)], gsems[b])

        def wait_gathers(b):
            pltpu.make_async_copy(x_hbm.at[pl.ds(0, 1024)],
                                  xrow.at[b], gsems[b]).wait()

        def fire_scatters(t, b):
            for j in range(CHUNK_ROWS):
                pltpu.async_copy(msg.at[b].at[pl.ds(j * 128, 128)],
                                 agg_sh.at[dst_buf.at[t % 4].at[j]],
                                 ssems[b], add=True)

        def wait_scatters(b):
            pltpu.make_async_copy(z_hbm.at[pl.ds(0, 1024)],
                                  msg.at[b], ssems[b]).wait()

        def compute(b):
            @pl.loop(0, 64, step=4)
            def _group(g0):
                for gg in range(4):
                    g = g0 + gg
                    rows = g * 16 + lanes
                    et_v = et_buf.at[b][pl.ds(g * 16, 16)]
                    if pairsum:
                        xs = [plsc.load_gather(xrow.at[b], [rows, cols[cc]])
                              for cc in range(6)]
                        for bb in range(3):
                            w0 = plsc.load_gather(wf_buf,
                                                  [et_v, cols[2 * bb]])
                            w1 = plsc.load_gather(wf_buf,
                                                  [et_v, cols[2 * bb + 1]])
                            plsc.store_scatter(
                                msg.at[b], [rows, cols[bb]],
                                xs[2 * bb] * w0 + xs[2 * bb + 1] * w1)
                    else:
                        xs = [plsc.load_gather(xrow.at[b], [rows, cols[cc]])
                              for cc in range(3)]
                        for k in range(6):
                            wk = plsc.load_gather(wf_buf, [et_v, cols[k]])
                            plsc.store_scatter(msg.at[b], [rows, cols[k]],
                                               xs[k >> 1] * wk)

        for h in load_idx(0, 0, isems[0]):
            h.wait()
        fire_gathers(0)
        load_idx(1, 1, isems[1])

        @pl.loop(0, N_CHUNKS // 2)
        def _step(u):
            for phase in range(2):
                t = u * 2 + phase
                b = phase
                nb = 1 - phase

                @pl.when(t + 1 < N_CHUNKS)
                def _():
                    wait_idx(nb)
                    fire_gathers(nb)

                wait_gathers(b)

                @pl.when(t >= 2)
                def _():
                    wait_scatters(b)

                compute(b)
                fire_scatters(t, b)

                @pl.when(t + 2 < N_CHUNKS)
                def _():
                    load_idx(t + 2, b, isems[b])

        wait_scatters(0)
        wait_scatters(1)
        plsc.subcore_barrier()
        pltpu.sync_copy(agg_sh.at[pl.ds(s * SLICE, SLICE)],
                        out_hbm.at[c, pl.ds(s * SLICE, SLICE)])

    return sc_agg


_sc_agg_mul = _make_sc_agg(pairsum=False)
_sc_agg_pair = _make_sc_agg(pairsum=True)


def _epi_mid(x_pad, agg2, rp, bp, mean: bool):
    def body(x_ref, a_ref, r_ref, b_ref, o_ref):
        x = x_ref[...]
        a = a_ref[0] + a_ref[1]
        core = jnp.dot(x, r_ref[...], preferred_element_type=jnp.float32)
        if mean:
            denom = jnp.maximum(a[:, 6:7], 1.0)
            agg6 = a[:, :6] / denom
        else:
            agg6 = a[:, :6]
        h = jnp.maximum(core + jnp.pad(agg6, ((0, 0), (0, 10))) + b_ref[...],
                        0.0)
        if mean:
            colid = jax.lax.broadcasted_iota(jnp.int32, (3136, 16), 1)
            h = jnp.where(colid == 6, denom, h)
        o_ref[...] = h

    return pl.pallas_call(
        body,
        grid=(N_PAD // 3136,),
        in_specs=[
            pl.BlockSpec((3136, 16), lambda i: (i, 0)),
            pl.BlockSpec((2, 3136, 8), lambda i: (0, i, 0)),
            pl.BlockSpec((16, 16), lambda i: (0, 0)),
            pl.BlockSpec((1, 16), lambda i: (0, 0)),
        ],
        out_specs=pl.BlockSpec((3136, 16), lambda i: (i, 0)),
        out_shape=jax.ShapeDtypeStruct((N_PAD, 16), jnp.float32),
    )(x_pad, agg2, rp, bp)


def _epi_final(x_pad, agg2, h1_pad, rp, bp):
    nblocks = N_PAD // 3136

    def body(x_ref, a_ref, d_ref, r_ref, b_ref, o_ref, acc_ref):
        i = pl.program_id(0)
        x = x_ref[...]
        a = a_ref[0] + a_ref[1]
        denom = d_ref[:, 6:7]
        core = jnp.dot(x, r_ref[...], preferred_element_type=jnp.float32)
        h = jnp.maximum(core[:, :6] + a[:, :6] / denom + b_ref[0, :6], 0.0)
        row = i * 3136 + jax.lax.broadcasted_iota(jnp.int32, (3136, 1), 0)
        h = jnp.where(row < N, h, 0.0)

        @pl.when(i == 0)
        def _():
            acc_ref[...] = jnp.zeros_like(acc_ref)

        acc_ref[...] += h

        @pl.when(i == nblocks - 1)
        def _():
            pooled = jnp.sum(acc_ref[...], axis=0, keepdims=True) / float(N)
            z = pooled - jnp.max(pooled, axis=1, keepdims=True)
            o_ref[...] = z - jnp.log(jnp.sum(jnp.exp(z), axis=1,
                                             keepdims=True))

    return pl.pallas_call(
        body,
        grid=(nblocks,),
        in_specs=[
            pl.BlockSpec((3136, 16), lambda i: (i, 0)),
            pl.BlockSpec((2, 3136, 8), lambda i: (0, i, 0)),
            pl.BlockSpec((3136, 16), lambda i: (i, 0)),
            pl.BlockSpec((16, 16), lambda i: (0, 0)),
            pl.BlockSpec((1, 16), lambda i: (0, 0)),
        ],
        out_specs=pl.BlockSpec((1, 6), lambda i: (0, 0)),
        out_shape=jax.ShapeDtypeStruct((1, 6), jnp.float32),
        scratch_shapes=[pltpu.VMEM((3136, 6), jnp.float32)],
    )(x_pad, agg2, h1_pad, rp, bp)


def _pad_wf(w):
    return jnp.zeros((R + 1, 8), jnp.float32).at[:R, :6].set(
        w.reshape(R, 6).astype(jnp.float32))


def _pad_root(r):
    return jnp.zeros((16, 16), jnp.float32).at[:r.shape[0], :r.shape[1]].set(r)


def _pad_bias(b):
    return jnp.zeros((1, 16), jnp.float32).at[0, :b.shape[0]].set(b)


def kernel(x, edge_index, batch, edge_type, w1, r1, b1, w2, r2, b2,
           w3, r3, b3):
    del batch
    src = edge_index[0]
    dst = edge_index[1]
    pad_e = E_PAD - E
    src_p = jnp.concatenate([src, jnp.zeros((pad_e,), jnp.int32)])
    et_p = jnp.concatenate([edge_type, jnp.full((pad_e,), R, jnp.int32)])
    dst_p = jnp.concatenate(
        [dst, jnp.full((pad_e,), N, jnp.int32)]).reshape(E_PAD // 128, 128)
    zeros8 = jnp.zeros((N_PAD, 8), jnp.float32)
    x0 = jnp.zeros((N_PAD, 16), jnp.float32).at[:N, :3].set(x)

    agg1 = _sc_agg_mul(x0, src_p, et_p, dst_p, _pad_wf(w1), zeros8)
    h1 = _epi_mid(x0, agg1, _pad_root(r1), _pad_bias(b1), mean=True)
    agg2 = _sc_agg_pair(h1, src_p, et_p, dst_p, _pad_wf(w2), zeros8)
    h2 = _epi_mid(h1, agg2, _pad_root(r2), _pad_bias(b2), mean=False)
    agg3 = _sc_agg_mul(h2, src_p, et_p, dst_p, _pad_wf(w3), zeros8)
    return _epi_final(h2, agg3, h1, _pad_root(r3), _pad_bias(b3))


# speedup vs baseline: 1.0637x; 1.0637x over previous
import dataclasses
import functools

import jax
import jax.numpy as jnp
from jax.experimental import pallas as pl
from jax.experimental.pallas import tpu as pltpu
from jax.experimental.pallas import tpu_sc as plsc

N = 50000
E = 1600000
R = 90

N_PAD = 50176
E_PAD = 1638400
ROWS_PER_TILE = E_PAD // 32 // 128
CHUNK_ROWS = 8
N_CHUNKS = ROWS_PER_TILE // CHUNK_ROWS
SLICE = N_PAD // 16


def _make_sc_agg(pairsum: bool):
    mesh = plsc.VectorSubcoreMesh(core_axis_name="c", subcore_axis_name="s")
    cp = pltpu.CompilerParams()
    for f, v in (("needs_layout_passes", False),
                 ("use_tc_tiling_on_sc", False)):
        if f in pltpu.CompilerParams.__dataclass_fields__:
            cp = dataclasses.replace(cp, **{f: v})

    @functools.partial(
        pl.kernel,
        compiler_params=cp,
        out_type=jax.ShapeDtypeStruct((2, N_PAD, 8), jnp.float32),
        mesh=mesh,
        scratch_types=[
            pltpu.VMEM((2, 1024), jnp.int32),
            pltpu.VMEM((2, 1024), jnp.int32),
            pltpu.VMEM((4, CHUNK_ROWS, 128), jnp.int32),
            pltpu.VMEM((2, 1024, 16), jnp.float32),
            pltpu.VMEM((2, 1024, 8), jnp.float32),
            pltpu.VMEM((R + 1, 8), jnp.float32),
            pltpu.VMEM_SHARED((N_PAD, 8), jnp.float32),
            pltpu.SemaphoreType.DMA,
            pltpu.SemaphoreType.DMA,
            pltpu.SemaphoreType.DMA,
            pltpu.SemaphoreType.DMA,
            pltpu.SemaphoreType.DMA,
            pltpu.SemaphoreType.DMA,
        ],
    )
    def sc_agg(x_hbm, src_hbm, et_hbm, dst_hbm, wf_hbm, z_hbm, out_hbm,
               src_buf, et_buf, dst_buf, xrow, msg, wf_buf, agg_sh,
               isem0, isem1, gsem0, gsem1, ssem0, ssem1):
        isems = (isem0, isem1)
        gsems = (gsem0, gsem1)
        ssems = (ssem0, ssem1)
        c = jax.lax.axis_index("c")
        s = jax.lax.axis_index("s")
        wid = c * 16 + s
        lanes = jax.lax.iota(jnp.int32, 16)
        cols = [jnp.full((16,), k, jnp.int32) for k in range(8)]
        zero16 = jnp.zeros((16,), jnp.float32)
        one16 = jnp.ones((16,), jnp.float32)
        base = wid * ROWS_PER_TILE

        pltpu.sync_copy(wf_hbm, wf_buf)
        pltpu.sync_copy(z_hbm.at[pl.ds(s * SLICE, SLICE)],
                        agg_sh.at[pl.ds(s * SLICE, SLICE)])

        @pl.loop(0, 64)
        def _init(g):
            rows = g * 16 + lanes
            for b in range(2):
                if pairsum:
                    for k in (3, 4, 5, 6, 7):
                        plsc.store_scatter(msg.at[b], [rows, cols[k]], zero16)
                else:
                    plsc.store_scatter(msg.at[b], [rows, cols[6]], one16)
                    plsc.store_scatter(msg.at[b], [rows, cols[7]], zero16)

        plsc.subcore_barrier()

        def load_idx(t, b, sem):
            r0 = base + t * CHUNK_ROWS
            e0 = r0 * 128
            h1 = pltpu.async_copy(src_hbm.at[pl.ds(e0, 1024)],
                                  src_buf.at[b], sem)
            h2 = pltpu.async_copy(et_hbm.at[pl.ds(e0, 1024)],
                                  et_buf.at[b], sem)
            h3 = pltpu.async_copy(dst_hbm.at[pl.ds(r0, CHUNK_ROWS)],
                                  dst_buf.at[t % 4], sem)
            return h1, h2, h3

        def wait_idx(b):
            pltpu.make_async_copy(src_hbm.at[pl.ds(0, 1024)],
                                  src_buf.at[b], isems[b]).wait()
            pltpu.make_async_copy(et_hbm.at[pl.ds(0, 1024)],
                                  et_buf.at[b], isems[b]).wait()
            pltpu.make_async_copy(dst_hbm.at[pl.ds(0, CHUNK_ROWS)],
                                  dst_buf.at[0], isems[b]).wait()

        def fire_gathers(b):
            for j in range(CHUNK_ROWS):
                pltpu.async_copy(
                    x_hbm.at[src_buf.at[b].at[pl.ds(j * 128, 128)]],
                    xrow.at[b].at[pl.ds(j * 128, 128)], gsems[b])

        def wait_gathers(b):
            pltpu.make_async_copy(x_hbm.at[pl.ds(0, 1024)],
                                  xrow.at[b], gsems[b]).wait()

        def fire_scatters(t, b):
            for j in range(CHUNK_ROWS):
                pltpu.async_copy(msg.at[b].at[pl.ds(j * 128, 128)],
                                 agg_sh.at[dst_buf.at[t % 4].at[j]],
                                 ssems[b], add=True)

        def wait_scatters(b):
            pltpu.make_async_copy(z_hbm.at[pl.ds(0, 1024)],
                                  msg.at[b], ssems[b]).wait()

        def compute(b):
            @plsc.parallel_loop(0, 64, unroll=4)
            def _group(g0):
                for gg in range(1):
                    g = g0 + gg
                    rows = g * 16 + lanes
                    et_v = et_buf.at[b][pl.ds(g * 16, 16)]
                    if pairsum:
                        xs = [plsc.load_gather(xrow.at[b], [rows, cols[cc]])
                              for cc in range(6)]
                        for bb in range(3):
                            w0 = plsc.load_gather(wf_buf,
                                                  [et_v, cols[2 * bb]])
                            w1 = plsc.load_gather(wf_buf,
                                                  [et_v, cols[2 * bb + 1]])
                            plsc.store_scatter(
                                msg.at[b], [rows, cols[bb]],
                                xs[2 * bb] * w0 + xs[2 * bb + 1] * w1)
                    else:
                        xs = [plsc.load_gather(xrow.at[b], [rows, cols[cc]])
                              for cc in range(3)]
                        for k in range(6):
                            wk = plsc.load_gather(wf_buf, [et_v, cols[k]])
                            plsc.store_scatter(msg.at[b], [rows, cols[k]],
                                               xs[k >> 1] * wk)

        for h in load_idx(0, 0, isems[0]):
            h.wait()
        fire_gathers(0)
        load_idx(1, 1, isems[1])

        @pl.loop(0, N_CHUNKS // 2)
        def _step(u):
            for phase in range(2):
                t = u * 2 + phase
                b = phase
                nb = 1 - phase

                @pl.when(t + 1 < N_CHUNKS)
                def _():
                    wait_idx(nb)
                    fire_gathers(nb)

                wait_gathers(b)

                @pl.when(t >= 2)
                def _():
                    wait_scatters(b)

                compute(b)
                fire_scatters(t, b)

                @pl.when(t + 2 < N_CHUNKS)
                def _():
                    load_idx(t + 2, b, isems[b])

        wait_scatters(0)
        wait_scatters(1)
        plsc.subcore_barrier()
        pltpu.sync_copy(agg_sh.at[pl.ds(s * SLICE, SLICE)],
                        out_hbm.at[c, pl.ds(s * SLICE, SLICE)])

    return sc_agg


_sc_agg_mul = _make_sc_agg(pairsum=False)
_sc_agg_pair = _make_sc_agg(pairsum=True)


def _epi_mid(x_pad, agg2, rp, bp, mean: bool):
    def body(x_ref, a_ref, r_ref, b_ref, o_ref):
        x = x_ref[...]
        a = a_ref[0] + a_ref[1]
        core = jnp.dot(x, r_ref[...], preferred_element_type=jnp.float32)
        if mean:
            denom = jnp.maximum(a[:, 6:7], 1.0)
            agg6 = a[:, :6] / denom
        else:
            agg6 = a[:, :6]
        h = jnp.maximum(core + jnp.pad(agg6, ((0, 0), (0, 10))) + b_ref[...],
                        0.0)
        if mean:
            colid = jax.lax.broadcasted_iota(jnp.int32, (3136, 16), 1)
            h = jnp.where(colid == 6, denom, h)
        o_ref[...] = h

    return pl.pallas_call(
        body,
        grid=(N_PAD // 3136,),
        in_specs=[
            pl.BlockSpec((3136, 16), lambda i: (i, 0)),
            pl.BlockSpec((2, 3136, 8), lambda i: (0, i, 0)),
            pl.BlockSpec((16, 16), lambda i: (0, 0)),
            pl.BlockSpec((1, 16), lambda i: (0, 0)),
        ],
        out_specs=pl.BlockSpec((3136, 16), lambda i: (i, 0)),
        out_shape=jax.ShapeDtypeStruct((N_PAD, 16), jnp.float32),
    )(x_pad, agg2, rp, bp)


def _epi_final(x_pad, agg2, h1_pad, rp, bp):
    nblocks = N_PAD // 3136

    def body(x_ref, a_ref, d_ref, r_ref, b_ref, o_ref, acc_ref):
        i = pl.program_id(0)
        x = x_ref[...]
        a = a_ref[0] + a_ref[1]
        denom = d_ref[:, 6:7]
        core = jnp.dot(x, r_ref[...], preferred_element_type=jnp.float32)
        h = jnp.maximum(core[:, :6] + a[:, :6] / denom + b_ref[0, :6], 0.0)
        row = i * 3136 + jax.lax.broadcasted_iota(jnp.int32, (3136, 1), 0)
        h = jnp.where(row < N, h, 0.0)

        @pl.when(i == 0)
        def _():
            acc_ref[...] = jnp.zeros_like(acc_ref)

        acc_ref[...] += h

        @pl.when(i == nblocks - 1)
        def _():
            pooled = jnp.sum(acc_ref[...], axis=0, keepdims=True) / float(N)
            z = pooled - jnp.max(pooled, axis=1, keepdims=True)
            o_ref[...] = z - jnp.log(jnp.sum(jnp.exp(z), axis=1,
                                             keepdims=True))

    return pl.pallas_call(
        body,
        grid=(nblocks,),
        in_specs=[
            pl.BlockSpec((3136, 16), lambda i: (i, 0)),
            pl.BlockSpec((2, 3136, 8), lambda i: (0, i, 0)),
            pl.BlockSpec((3136, 16), lambda i: (i, 0)),
            pl.BlockSpec((16, 16), lambda i: (0, 0)),
            pl.BlockSpec((1, 16), lambda i: (0, 0)),
        ],
        out_specs=pl.BlockSpec((1, 6), lambda i: (0, 0)),
        out_shape=jax.ShapeDtypeStruct((1, 6), jnp.float32),
        scratch_shapes=[pltpu.VMEM((3136, 6), jnp.float32)],
    )(x_pad, agg2, h1_pad, rp, bp)


def _pad_wf(w):
    return jnp.zeros((R + 1, 8), jnp.float32).at[:R, :6].set(
        w.reshape(R, 6).astype(jnp.float32))


def _pad_root(r):
    return jnp.zeros((16, 16), jnp.float32).at[:r.shape[0], :r.shape[1]].set(r)


def _pad_bias(b):
    return jnp.zeros((1, 16), jnp.float32).at[0, :b.shape[0]].set(b)


def kernel(x, edge_index, batch, edge_type, w1, r1, b1, w2, r2, b2,
           w3, r3, b3):
    del batch
    src = edge_index[0]
    dst = edge_index[1]
    pad_e = E_PAD - E
    src_p = jnp.concatenate([src, jnp.zeros((pad_e,), jnp.int32)])
    et_p = jnp.concatenate([edge_type, jnp.full((pad_e,), R, jnp.int32)])
    dst_p = jnp.concatenate(
        [dst, jnp.full((pad_e,), N, jnp.int32)]).reshape(E_PAD // 128, 128)
    zeros8 = jnp.zeros((N_PAD, 8), jnp.float32)
    x0 = jnp.zeros((N_PAD, 16), jnp.float32).at[:N, :3].set(x)

    agg1 = _sc_agg_mul(x0, src_p, et_p, dst_p, _pad_wf(w1), zeros8)
    h1 = _epi_mid(x0, agg1, _pad_root(r1), _pad_bias(b1), mean=True)
    agg2 = _sc_agg_pair(h1, src_p, et_p, dst_p, _pad_wf(w2), zeros8)
    h2 = _epi_mid(h1, agg2, _pad_root(r2), _pad_bias(b2), mean=False)
    agg3 = _sc_agg_mul(h2, src_p, et_p, dst_p, _pad_wf(w3), zeros8)
    return _epi_final(h2, agg3, h1, _pad_root(r3), _pad_bias(b3))


# speedup vs baseline: 1.7805x; 1.6739x over previous
import dataclasses
import functools

import jax
import jax.numpy as jnp
from jax.experimental import pallas as pl
from jax.experimental.pallas import tpu as pltpu
from jax.experimental.pallas import tpu_sc as plsc

N = 50000
E = 1600000
R = 90

N_PAD = 50176
E_PAD = 1638400
ROWS_PER_TILE = E_PAD // 32 // 128
CHUNK_ROWS = 8
N_CHUNKS = ROWS_PER_TILE // CHUNK_ROWS
SLICE = N_PAD // 16


def _make_sc_agg(pairsum: bool):
    mesh = plsc.VectorSubcoreMesh(core_axis_name="c", subcore_axis_name="s")
    cp = pltpu.CompilerParams()
    for f, v in (("needs_layout_passes", False),
                 ("use_tc_tiling_on_sc", False)):
        if f in pltpu.CompilerParams.__dataclass_fields__:
            cp = dataclasses.replace(cp, **{f: v})

    @functools.partial(
        pl.kernel,
        compiler_params=cp,
        out_type=jax.ShapeDtypeStruct((2, N_PAD, 8), jnp.float32),
        mesh=mesh,
        scratch_types=[
            pltpu.VMEM((2, 1024), jnp.int32),
            pltpu.VMEM((2, 1024), jnp.int32),
            pltpu.VMEM((4, CHUNK_ROWS, 128), jnp.int32),
            pltpu.VMEM((2, 1024, 8), jnp.float32),
            pltpu.VMEM((2, 1024, 8), jnp.float32),
            pltpu.VMEM((R + 1, 8), jnp.float32),
            pltpu.VMEM_SHARED((N_PAD, 8), jnp.float32),
            pltpu.VMEM_SHARED((N_PAD, 8), jnp.float32),
            pltpu.SemaphoreType.DMA,
            pltpu.SemaphoreType.DMA,
            pltpu.SemaphoreType.DMA,
            pltpu.SemaphoreType.DMA,
            pltpu.SemaphoreType.DMA,
            pltpu.SemaphoreType.DMA,
        ],
    )
    def sc_agg(x_hbm, src_hbm, et_hbm, dst_hbm, wf_hbm, z_hbm, out_hbm,
               src_buf, et_buf, dst_buf, xrow, msg, wf_buf, agg_sh, x_sh,
               isem0, isem1, gsem0, gsem1, ssem0, ssem1):
        isems = (isem0, isem1)
        gsems = (gsem0, gsem1)
        ssems = (ssem0, ssem1)
        c = jax.lax.axis_index("c")
        s = jax.lax.axis_index("s")
        wid = c * 16 + s
        lanes = jax.lax.iota(jnp.int32, 16)
        cols = [jnp.full((16,), k, jnp.int32) for k in range(8)]
        zero16 = jnp.zeros((16,), jnp.float32)
        one16 = jnp.ones((16,), jnp.float32)
        base = wid * ROWS_PER_TILE

        pltpu.sync_copy(wf_hbm, wf_buf)
        pltpu.sync_copy(z_hbm.at[pl.ds(s * SLICE, SLICE)],
                        agg_sh.at[pl.ds(s * SLICE, SLICE)])
        pltpu.sync_copy(x_hbm.at[pl.ds(s * SLICE, SLICE)],
                        x_sh.at[pl.ds(s * SLICE, SLICE)])

        @pl.loop(0, 64)
        def _init(g):
            rows = g * 16 + lanes
            for b in range(2):
                if pairsum:
                    for k in (3, 4, 5, 6, 7):
                        plsc.store_scatter(msg.at[b], [rows, cols[k]], zero16)
                else:
                    plsc.store_scatter(msg.at[b], [rows, cols[6]], one16)
                    plsc.store_scatter(msg.at[b], [rows, cols[7]], zero16)

        plsc.subcore_barrier()

        def load_idx(t, b, sem):
            r0 = base + t * CHUNK_ROWS
            e0 = r0 * 128
            h1 = pltpu.async_copy(src_hbm.at[pl.ds(e0, 1024)],
                                  src_buf.at[b], sem)
            h2 = pltpu.async_copy(et_hbm.at[pl.ds(e0, 1024)],
                                  et_buf.at[b], sem)
            h3 = pltpu.async_copy(dst_hbm.at[pl.ds(r0, CHUNK_ROWS)],
                                  dst_buf.at[t % 4], sem)
            return h1, h2, h3

        def wait_idx(b):
            pltpu.make_async_copy(src_hbm.at[pl.ds(0, 1024)],
                                  src_buf.at[b], isems[b]).wait()
            pltpu.make_async_copy(et_hbm.at[pl.ds(0, 1024)],
                                  et_buf.at[b], isems[b]).wait()
            pltpu.make_async_copy(dst_hbm.at[pl.ds(0, CHUNK_ROWS)],
                                  dst_buf.at[0], isems[b]).wait()

        def fire_gathers(b):
            for j in range(CHUNK_ROWS):
                pltpu.async_copy(
                    x_sh.at[src_buf.at[b].at[pl.ds(j * 128, 128)]],
                    xrow.at[b].at[pl.ds(j * 128, 128)], gsems[b])

        def wait_gathers(b):
            pltpu.make_async_copy(x_hbm.at[pl.ds(0, 1024)],
                                  xrow.at[b], gsems[b]).wait()

        def fire_scatters(t, b):
            for j in range(CHUNK_ROWS):
                pltpu.async_copy(msg.at[b].at[pl.ds(j * 128, 128)],
                                 agg_sh.at[dst_buf.at[t % 4].at[j]],
                                 ssems[b], add=True)

        def wait_scatters(b):
            pltpu.make_async_copy(z_hbm.at[pl.ds(0, 1024)],
                                  msg.at[b], ssems[b]).wait()

        def compute(b):
            @plsc.parallel_loop(0, 64, unroll=4)
            def _group(g0):
                for gg in range(1):
                    g = g0 + gg
                    rows = g * 16 + lanes
                    et_v = et_buf.at[b][pl.ds(g * 16, 16)]
                    if pairsum:
                        xs = [plsc.load_gather(xrow.at[b], [rows, cols[cc]])
                              for cc in range(6)]
                        for bb in range(3):
                            w0 = plsc.load_gather(wf_buf,
                                                  [et_v, cols[2 * bb]])
                            w1 = plsc.load_gather(wf_buf,
                                                  [et_v, cols[2 * bb + 1]])
                            plsc.store_scatter(
                                msg.at[b], [rows, cols[bb]],
                                xs[2 * bb] * w0 + xs[2 * bb + 1] * w1)
                    else:
                        xs = [plsc.load_gather(xrow.at[b], [rows, cols[cc]])
                              for cc in range(3)]
                        for k in range(6):
                            wk = plsc.load_gather(wf_buf, [et_v, cols[k]])
                            plsc.store_scatter(msg.at[b], [rows, cols[k]],
                                               xs[k >> 1] * wk)

        for h in load_idx(0, 0, isems[0]):
            h.wait()
        fire_gathers(0)
        load_idx(1, 1, isems[1])

        @pl.loop(0, N_CHUNKS // 2)
        def _step(u):
            for phase in range(2):
                t = u * 2 + phase
                b = phase
                nb = 1 - phase

                @pl.when(t + 1 < N_CHUNKS)
                def _():
                    wait_idx(nb)
                    fire_gathers(nb)

                wait_gathers(b)

                @pl.when(t >= 2)
                def _():
                    wait_scatters(b)

                compute(b)
                fire_scatters(t, b)

                @pl.when(t + 2 < N_CHUNKS)
                def _():
                    load_idx(t + 2, b, isems[b])

        wait_scatters(0)
        wait_scatters(1)
        plsc.subcore_barrier()
        pltpu.sync_copy(agg_sh.at[pl.ds(s * SLICE, SLICE)],
                        out_hbm.at[c, pl.ds(s * SLICE, SLICE)])

    return sc_agg


_sc_agg_mul = _make_sc_agg(pairsum=False)
_sc_agg_pair = _make_sc_agg(pairsum=True)


def _epi_mid(x_pad, agg2, rp, bp, mean: bool):
    def body(x_ref, a_ref, r_ref, b_ref, o_ref):
        x = x_ref[...]
        a = a_ref[0] + a_ref[1]
        core = jnp.dot(x, r_ref[...], preferred_element_type=jnp.float32)
        if mean:
            denom = jnp.maximum(a[:, 6:7], 1.0)
            agg6 = a[:, :6] / denom
        else:
            agg6 = a[:, :6]
        h = jnp.maximum(core + jnp.pad(agg6, ((0, 0), (0, 2))) + b_ref[...],
                        0.0)
        if mean:
            colid = jax.lax.broadcasted_iota(jnp.int32, (3136, 8), 1)
            h = jnp.where(colid == 6, denom, h)
        o_ref[...] = h

    return pl.pallas_call(
        body,
        grid=(N_PAD // 3136,),
        in_specs=[
            pl.BlockSpec((3136, 8), lambda i: (i, 0)),
            pl.BlockSpec((2, 3136, 8), lambda i: (0, i, 0)),
            pl.BlockSpec((8, 8), lambda i: (0, 0)),
            pl.BlockSpec((1, 8), lambda i: (0, 0)),
        ],
        out_specs=pl.BlockSpec((3136, 8), lambda i: (i, 0)),
        out_shape=jax.ShapeDtypeStruct((N_PAD, 8), jnp.float32),
    )(x_pad, agg2, rp, bp)


def _epi_final(x_pad, agg2, h1_pad, rp, bp):
    nblocks = N_PAD // 3136

    def body(x_ref, a_ref, d_ref, r_ref, b_ref, o_ref, acc_ref):
        i = pl.program_id(0)
        x = x_ref[...]
        a = a_ref[0] + a_ref[1]
        denom = d_ref[:, 6:7]
        core = jnp.dot(x, r_ref[...], preferred_element_type=jnp.float32)
        h = jnp.maximum(core[:, :6] + a[:, :6] / denom + b_ref[0, :6], 0.0)
        row = i * 3136 + jax.lax.broadcasted_iota(jnp.int32, (3136, 1), 0)
        h = jnp.where(row < N, h, 0.0)

        @pl.when(i == 0)
        def _():
            acc_ref[...] = jnp.zeros_like(acc_ref)

        acc_ref[...] += h

        @pl.when(i == nblocks - 1)
        def _():
            pooled = jnp.sum(acc_ref[...], axis=0, keepdims=True) / float(N)
            z = pooled - jnp.max(pooled, axis=1, keepdims=True)
            o_ref[...] = z - jnp.log(jnp.sum(jnp.exp(z), axis=1,
                                             keepdims=True))

    return pl.pallas_call(
        body,
        grid=(nblocks,),
        in_specs=[
            pl.BlockSpec((3136, 8), lambda i: (i, 0)),
            pl.BlockSpec((2, 3136, 8), lambda i: (0, i, 0)),
            pl.BlockSpec((3136, 8), lambda i: (i, 0)),
            pl.BlockSpec((8, 8), lambda i: (0, 0)),
            pl.BlockSpec((1, 8), lambda i: (0, 0)),
        ],
        out_specs=pl.BlockSpec((1, 6), lambda i: (0, 0)),
        out_shape=jax.ShapeDtypeStruct((1, 6), jnp.float32),
        scratch_shapes=[pltpu.VMEM((3136, 6), jnp.float32)],
    )(x_pad, agg2, h1_pad, rp, bp)


def _pad_wf(w):
    return jnp.zeros((R + 1, 8), jnp.float32).at[:R, :6].set(
        w.reshape(R, 6).astype(jnp.float32))


def _pad_root(r):
    return jnp.zeros((8, 8), jnp.float32).at[:r.shape[0], :r.shape[1]].set(r)


def _pad_bias(b):
    return jnp.zeros((1, 8), jnp.float32).at[0, :b.shape[0]].set(b)


def kernel(x, edge_index, batch, edge_type, w1, r1, b1, w2, r2, b2,
           w3, r3, b3):
    del batch
    src = edge_index[0]
    dst = edge_index[1]
    pad_e = E_PAD - E
    src_p = jnp.concatenate([src, jnp.zeros((pad_e,), jnp.int32)])
    et_p = jnp.concatenate([edge_type, jnp.full((pad_e,), R, jnp.int32)])
    dst_p = jnp.concatenate(
        [dst, jnp.full((pad_e,), N, jnp.int32)]).reshape(E_PAD // 128, 128)
    zeros8 = jnp.zeros((N_PAD, 8), jnp.float32)
    x0 = jnp.zeros((N_PAD, 8), jnp.float32).at[:N, :3].set(x)

    agg1 = _sc_agg_mul(x0, src_p, et_p, dst_p, _pad_wf(w1), zeros8)
    h1 = _epi_mid(x0, agg1, _pad_root(r1), _pad_bias(b1), mean=True)
    agg2 = _sc_agg_pair(h1, src_p, et_p, dst_p, _pad_wf(w2), zeros8)
    h2 = _epi_mid(h1, agg2, _pad_root(r2), _pad_bias(b2), mean=False)
    agg3 = _sc_agg_mul(h2, src_p, et_p, dst_p, _pad_wf(w3), zeros8)
    return _epi_final(h2, agg3, h1, _pad_root(r3), _pad_bias(b3))


# speedup vs baseline: 2.0393x; 1.1453x over previous
import dataclasses
import functools

import jax
import jax.numpy as jnp
from jax.experimental import pallas as pl
from jax.experimental.pallas import tpu as pltpu
from jax.experimental.pallas import tpu_sc as plsc

N = 50000
E = 1600000
R = 90

N_PAD = 50176
E_PAD = 1638400
ROWS_PER_TILE = E_PAD // 32 // 128
CHUNK_ROWS = 8
N_CHUNKS = ROWS_PER_TILE // CHUNK_ROWS
SLICE = N_PAD // 16


def _make_sc_agg(pairsum: bool, epi=None):
    mesh = plsc.VectorSubcoreMesh(core_axis_name="c", subcore_axis_name="s")
    cp = pltpu.CompilerParams()
    for f, v in (("needs_layout_passes", False),
                 ("use_tc_tiling_on_sc", False)):
        if f in pltpu.CompilerParams.__dataclass_fields__:
            cp = dataclasses.replace(cp, **{f: v})

    agg_sds = jax.ShapeDtypeStruct((2, N_PAD, 8), jnp.float32)
    h_sds = jax.ShapeDtypeStruct((N_PAD, 8), jnp.float32)
    scratch = [
        pltpu.VMEM((2, 1024), jnp.int32),
        pltpu.VMEM((2, 1024), jnp.int32),
        pltpu.VMEM((4, CHUNK_ROWS, 128), jnp.int32),
        pltpu.VMEM((2, 1024, 8), jnp.float32),
        pltpu.VMEM((2, 1024, 8), jnp.float32),
        pltpu.VMEM((R + 1, 8), jnp.float32),
        pltpu.VMEM_SHARED((N_PAD, 8), jnp.float32),
        pltpu.VMEM_SHARED((N_PAD, 8), jnp.float32),
        pltpu.SemaphoreType.DMA,
        pltpu.SemaphoreType.DMA,
        pltpu.SemaphoreType.DMA,
        pltpu.SemaphoreType.DMA,
        pltpu.SemaphoreType.DMA,
        pltpu.SemaphoreType.DMA,
    ]
    if epi is not None:
        scratch += [
            pltpu.VMEM((SLICE // 4, 8), jnp.float32),
            pltpu.VMEM((SLICE // 4, 8), jnp.float32),
            pltpu.VMEM((SLICE // 4, 8), jnp.float32),
            pltpu.VMEM((64,), jnp.float32),
            pltpu.VMEM((16,), jnp.float32),
        ]

    @functools.partial(
        pl.kernel,
        compiler_params=cp,
        out_type=agg_sds if epi is None else (agg_sds, h_sds),
        mesh=mesh,
        scratch_types=scratch,
    )
    def sc_agg(*refs):
        if epi is None:
            (x_hbm, src_hbm, et_hbm, dst_hbm, wf_hbm, z_hbm, out_hbm,
             src_buf, et_buf, dst_buf, xrow, msg, wf_buf, agg_sh, x_sh,
             isem0, isem1, gsem0, gsem1, ssem0, ssem1) = refs
        else:
            (src_hbm, et_hbm, dst_hbm, wf_hbm, z_hbm, agg_in, x0_in,
             r_hbm, b_hbm, out_hbm, h_out,
             src_buf, et_buf, dst_buf, xrow, msg, wf_buf, agg_sh, x_sh,
             isem0, isem1, gsem0, gsem1, ssem0, ssem1,
             p0buf, p1buf, x0buf, rbuf, bbuf) = refs
        isems = (isem0, isem1)
        gsems = (gsem0, gsem1)
        ssems = (ssem0, ssem1)
        c = jax.lax.axis_index("c")
        s = jax.lax.axis_index("s")
        wid = c * 16 + s
        lanes = jax.lax.iota(jnp.int32, 16)
        cols = [jnp.full((16,), k, jnp.int32) for k in range(8)]
        zero16 = jnp.zeros((16,), jnp.float32)
        one16 = jnp.ones((16,), jnp.float32)
        base = wid * ROWS_PER_TILE
        sl = pl.ds(s * SLICE, SLICE)

        pltpu.sync_copy(wf_hbm, wf_buf)
        pltpu.sync_copy(z_hbm.at[sl], agg_sh.at[sl])

        if epi is None:
            pltpu.sync_copy(x_hbm.at[sl], x_sh.at[sl])
        else:
            mean_flag, in_cols, out_cols = epi
            pltpu.sync_copy(r_hbm, rbuf)
            pltpu.sync_copy(b_hbm, bbuf.at[pl.ds(0, 8)])
            rvecs = [rbuf[pl.ds(o * 16, 16)] for o in range(4)]
            rsc = [[rvecs[(i * 8 + k) // 16][(i * 8 + k) % 16]
                    for k in range(out_cols)] for i in range(in_cols)]
            bvec = bbuf[pl.ds(0, 16)]
            bsc = [bvec[k] for k in range(out_cols)]
            qr = SLICE // 4
            for q in range(4):
                off = pl.ds(s * SLICE + q * qr, qr)
                pltpu.sync_copy(agg_in.at[0, off], p0buf)
                pltpu.sync_copy(agg_in.at[1, off], p1buf)
                pltpu.sync_copy(x0_in.at[off], x0buf)

                @plsc.parallel_loop(0, qr // 16, unroll=7)
                def _epi(g):
                    rows = g * 16 + lanes
                    xs = [plsc.load_gather(x0buf, [rows, cols[i]])
                          for i in range(in_cols)]
                    if mean_flag:
                        cnt = (plsc.load_gather(p0buf, [rows, cols[6]])
                               + plsc.load_gather(p1buf, [rows, cols[6]]))
                        den = jnp.maximum(cnt, 1.0)
                        inv = 1.0 / den
                    for k in range(out_cols):
                        a = (plsc.load_gather(p0buf, [rows, cols[k]])
                             + plsc.load_gather(p1buf, [rows, cols[k]]))
                        if mean_flag:
                            a = a * inv
                        for i in range(in_cols):
                            a = a + xs[i] * rsc[i][k]
                        a = jnp.maximum(a + bsc[k], 0.0)
                        plsc.store_scatter(p0buf, [rows, cols[k]], a)
                    if mean_flag:
                        plsc.store_scatter(p0buf, [rows, cols[6]], den)

                pltpu.sync_copy(p0buf, x_sh.at[off])

                @pl.when(c == 0)
                def _():
                    pltpu.sync_copy(p0buf, h_out.at[off])

        @pl.loop(0, 64)
        def _init(g):
            rows = g * 16 + lanes
            for b in range(2):
                if pairsum:
                    for k in (3, 4, 5, 6, 7):
                        plsc.store_scatter(msg.at[b], [rows, cols[k]], zero16)
                else:
                    plsc.store_scatter(msg.at[b], [rows, cols[6]], one16)
                    plsc.store_scatter(msg.at[b], [rows, cols[7]], zero16)

        plsc.subcore_barrier()

        def load_idx(t, b, sem):
            r0 = base + t * CHUNK_ROWS
            e0 = r0 * 128
            h1 = pltpu.async_copy(src_hbm.at[pl.ds(e0, 1024)],
                                  src_buf.at[b], sem)
            h2 = pltpu.async_copy(et_hbm.at[pl.ds(e0, 1024)],
                                  et_buf.at[b], sem)
            h3 = pltpu.async_copy(dst_hbm.at[pl.ds(r0, CHUNK_ROWS)],
                                  dst_buf.at[t % 4], sem)
            return h1, h2, h3

        def wait_idx(b):
            pltpu.make_async_copy(src_hbm.at[pl.ds(0, 1024)],
                                  src_buf.at[b], isems[b]).wait()
            pltpu.make_async_copy(et_hbm.at[pl.ds(0, 1024)],
                                  et_buf.at[b], isems[b]).wait()
            pltpu.make_async_copy(dst_hbm.at[pl.ds(0, CHUNK_ROWS)],
                                  dst_buf.at[0], isems[b]).wait()

        def fire_gathers(b):
            for j in range(CHUNK_ROWS):
                pltpu.async_copy(
                    x_sh.at[src_buf.at[b].at[pl.ds(j * 128, 128)]],
                    xrow.at[b].at[pl.ds(j * 128, 128)], gsems[b])

        def wait_gathers(b):
            pltpu.make_async_copy(z_hbm.at[pl.ds(0, 1024)],
                                  xrow.at[b], gsems[b]).wait()

        def fire_scatters(t, b):
            for j in range(CHUNK_ROWS):
                pltpu.async_copy(msg.at[b].at[pl.ds(j * 128, 128)],
                                 agg_sh.at[dst_buf.at[t % 4].at[j]],
                                 ssems[b], add=True)

        def wait_scatters(b):
            pltpu.make_async_copy(z_hbm.at[pl.ds(0, 1024)],
                                  msg.at[b], ssems[b]).wait()

        def compute(b):
            @plsc.parallel_loop(0, 64, unroll=4)
            def _group(g):
                rows = g * 16 + lanes
                et_v = et_buf.at[b][pl.ds(g * 16, 16)]
                if pairsum:
                    xs = [plsc.load_gather(xrow.at[b], [rows, cols[cc]])
                          for cc in range(6)]
                    for bb in range(3):
                        w0 = plsc.load_gather(wf_buf,
                                              [et_v, cols[2 * bb]])
                        w1 = plsc.load_gather(wf_buf,
                                              [et_v, cols[2 * bb + 1]])
                        plsc.store_scatter(
                            msg.at[b], [rows, cols[bb]],
                            xs[2 * bb] * w0 + xs[2 * bb + 1] * w1)
                else:
                    xs = [plsc.load_gather(xrow.at[b], [rows, cols[cc]])
                          for cc in range(3)]
                    for k in range(6):
                        wk = plsc.load_gather(wf_buf, [et_v, cols[k]])
                        plsc.store_scatter(msg.at[b], [rows, cols[k]],
                                           xs[k >> 1] * wk)

        for h in load_idx(0, 0, isems[0]):
            h.wait()
        fire_gathers(0)
        load_idx(1, 1, isems[1])

        @pl.loop(0, N_CHUNKS // 2)
        def _step(u):
            for phase in range(2):
                t = u * 2 + phase
                b = phase
                nb = 1 - phase

                @pl.when(t + 1 < N_CHUNKS)
                def _():
                    wait_idx(nb)
                    fire_gathers(nb)

                wait_gathers(b)

                @pl.when(t >= 2)
                def _():
                    wait_scatters(b)

                compute(b)
                fire_scatters(t, b)

                @pl.when(t + 2 < N_CHUNKS)
                def _():
                    load_idx(t + 2, b, isems[b])

        wait_scatters(0)
        wait_scatters(1)
        plsc.subcore_barrier()
        pltpu.sync_copy(agg_sh.at[sl], out_hbm.at[c, sl])

    return sc_agg


_sc_l1 = _make_sc_agg(pairsum=False)
_sc_l2 = _make_sc_agg(pairsum=True, epi=(True, 3, 6))
_sc_l3 = _make_sc_agg(pairsum=False, epi=(False, 6, 3))


def _epi_final(x_pad, agg2, h1_pad, rp, bp):
    nblocks = N_PAD // 3136

    def body(x_ref, a_ref, d_ref, r_ref, b_ref, o_ref, acc_ref):
        i = pl.program_id(0)
        x = x_ref[...]
        a = a_ref[0] + a_ref[1]
        denom = d_ref[:, 6:7]
        core = jnp.dot(x, r_ref[...], preferred_element_type=jnp.float32)
        h = jnp.maximum(core[:, :6] + a[:, :6] / denom + b_ref[0, :6], 0.0)
        row = i * 3136 + jax.lax.broadcasted_iota(jnp.int32, (3136, 1), 0)
        h = jnp.where(row < N, h, 0.0)

        @pl.when(i == 0)
        def _():
            acc_ref[...] = jnp.zeros_like(acc_ref)

        acc_ref[...] += h

        @pl.when(i == nblocks - 1)
        def _():
            pooled = jnp.sum(acc_ref[...], axis=0, keepdims=True) / float(N)
            z = pooled - jnp.max(pooled, axis=1, keepdims=True)
            o_ref[...] = z - jnp.log(jnp.sum(jnp.exp(z), axis=1,
                                             keepdims=True))

    return pl.pallas_call(
        body,
        grid=(nblocks,),
        in_specs=[
            pl.BlockSpec((3136, 8), lambda i: (i, 0)),
            pl.BlockSpec((2, 3136, 8), lambda i: (0, i, 0)),
            pl.BlockSpec((3136, 8), lambda i: (i, 0)),
            pl.BlockSpec((8, 8), lambda i: (0, 0)),
            pl.BlockSpec((1, 8), lambda i: (0, 0)),
        ],
        out_specs=pl.BlockSpec((1, 6), lambda i: (0, 0)),
        out_shape=jax.ShapeDtypeStruct((1, 6), jnp.float32),
        scratch_shapes=[pltpu.VMEM((3136, 6), jnp.float32)],
    )(x_pad, agg2, h1_pad, rp, bp)


def _pad_wf(w):
    return jnp.zeros((R + 1, 8), jnp.float32).at[:R, :6].set(
        w.reshape(R, 6).astype(jnp.float32))


def _pad_root(r):
    return jnp.zeros((8, 8), jnp.float32).at[:r.shape[0], :r.shape[1]].set(r)


def _pad_bias(b):
    return jnp.zeros((1, 8), jnp.float32).at[0, :b.shape[0]].set(b)


def kernel(x, edge_index, batch, edge_type, w1, r1, b1, w2, r2, b2,
           w3, r3, b3):
    del batch
    src = edge_index[0]
    dst = edge_index[1]
    pad_e = E_PAD - E
    src_p = jnp.concatenate([src, jnp.zeros((pad_e,), jnp.int32)])
    et_p = jnp.concatenate([edge_type, jnp.full((pad_e,), R, jnp.int32)])
    dst_p = jnp.concatenate(
        [dst, jnp.full((pad_e,), N, jnp.int32)]).reshape(E_PAD // 128, 128)
    zeros8 = jnp.zeros((N_PAD, 8), jnp.float32)
    x0 = jnp.zeros((N_PAD, 8), jnp.float32).at[:N, :3].set(x)

    agg1 = _sc_l1(x0, src_p, et_p, dst_p, _pad_wf(w1), zeros8)
    agg2, h1 = _sc_l2(src_p, et_p, dst_p, _pad_wf(w2), zeros8,
                      agg1, x0, _pad_root(r1).reshape(64),
                      jnp.zeros((8,), jnp.float32).at[:6].set(b1))
    agg3, h2 = _sc_l3(src_p, et_p, dst_p, _pad_wf(w3), zeros8,
                      agg2, h1, _pad_root(r2).reshape(64),
                      jnp.zeros((8,), jnp.float32).at[:3].set(b2))
    return _epi_final(h2, agg3, h1, _pad_root(r3), _pad_bias(b3))


# speedup vs baseline: 2.2083x; 1.0829x over previous
import dataclasses
import functools

import jax
import jax.numpy as jnp
from jax.experimental import pallas as pl
from jax.experimental.pallas import tpu as pltpu
from jax.experimental.pallas import tpu_sc as plsc

N = 50000
E = 1600000
R = 90

N_PAD = 50176
E_ROWS = E // 128
CHUNK_ROWS = 8
MAIN_CHUNKS = 48
SLICE = N_PAD // 16


def _make_sc_agg(pairsum: bool, epi=None):
    mesh = plsc.VectorSubcoreMesh(core_axis_name="c", subcore_axis_name="s")
    cp = pltpu.CompilerParams()
    for f, v in (("needs_layout_passes", False),
                 ("use_tc_tiling_on_sc", False)):
        if f in pltpu.CompilerParams.__dataclass_fields__:
            cp = dataclasses.replace(cp, **{f: v})

    agg_sds = jax.ShapeDtypeStruct((2, N_PAD, 8), jnp.float32)
    h_sds = jax.ShapeDtypeStruct((N_PAD, 8), jnp.float32)
    scratch = [
        pltpu.VMEM((2, 1024), jnp.int32),
        pltpu.VMEM((2, 1024), jnp.int32),
        pltpu.VMEM((4, CHUNK_ROWS, 128), jnp.int32),
        pltpu.VMEM((2, 1024, 8), jnp.float32),
        pltpu.VMEM((2, 1024, 8), jnp.float32),
        pltpu.VMEM((R + 1, 8), jnp.float32),
        pltpu.VMEM_SHARED((N_PAD, 8), jnp.float32),
        pltpu.VMEM_SHARED((N_PAD, 8), jnp.float32),
        pltpu.SemaphoreType.DMA,
        pltpu.SemaphoreType.DMA,
        pltpu.SemaphoreType.DMA,
        pltpu.SemaphoreType.DMA,
        pltpu.SemaphoreType.DMA,
        pltpu.SemaphoreType.DMA,
    ]
    if epi is not None:
        scratch += [
            pltpu.VMEM((SLICE // 4, 8), jnp.float32),
            pltpu.VMEM((SLICE // 4, 8), jnp.float32),
            pltpu.VMEM((SLICE // 4, 8), jnp.float32),
            pltpu.VMEM((64,), jnp.float32),
            pltpu.VMEM((16,), jnp.float32),
        ]

    @functools.partial(
        pl.kernel,
        compiler_params=cp,
        out_type=agg_sds if epi is None else (agg_sds, h_sds),
        mesh=mesh,
        scratch_types=scratch,
    )
    def sc_agg(*refs):
        if epi is None:
            (x_hbm, ei_hbm, et_hbm, wf_hbm, z_hbm, out_hbm,
             src_buf, et_buf, dst_buf, xrow, msg, wf_buf, agg_sh, x_sh,
             isem0, isem1, gsem0, gsem1, ssem0, ssem1) = refs
        else:
            (ei_hbm, et_hbm, wf_hbm, z_hbm, agg_in, x0_in,
             r_hbm, b_hbm, out_hbm, h_out,
             src_buf, et_buf, dst_buf, xrow, msg, wf_buf, agg_sh, x_sh,
             isem0, isem1, gsem0, gsem1, ssem0, ssem1,
             p0buf, p1buf, x0buf, rbuf, bbuf) = refs
        isems = (isem0, isem1)
        gsems = (gsem0, gsem1)
        ssems = (ssem0, ssem1)
        c = jax.lax.axis_index("c")
        s = jax.lax.axis_index("s")
        wid = c * 16 + s
        lanes = jax.lax.iota(jnp.int32, 16)
        cols = [jnp.full((16,), k, jnp.int32) for k in range(8)]
        zero16 = jnp.zeros((16,), jnp.float32)
        one16 = jnp.ones((16,), jnp.float32)
        base = wid * 390 + jnp.minimum(wid, 20)
        sl = pl.ds(s * SLICE, SLICE)

        pltpu.sync_copy(wf_hbm, wf_buf)
        pltpu.sync_copy(z_hbm.at[sl], agg_sh.at[sl])

        if epi is None:
            pltpu.sync_copy(x_hbm.at[sl], x_sh.at[sl])
        else:
            mean_flag, in_cols, out_cols = epi
            pltpu.sync_copy(r_hbm, rbuf)
            pltpu.sync_copy(b_hbm, bbuf.at[pl.ds(0, 8)])
            rvecs = [rbuf[pl.ds(o * 16, 16)] for o in range(4)]
            rsc = [[rvecs[(i * 8 + k) // 16][(i * 8 + k) % 16]
                    for k in range(out_cols)] for i in range(in_cols)]
            bvec = bbuf[pl.ds(0, 16)]
            bsc = [bvec[k] for k in range(out_cols)]
            qr = SLICE // 4
            for q in range(4):
                off = pl.ds(s * SLICE + q * qr, qr)
                pltpu.sync_copy(agg_in.at[0, off], p0buf)
                pltpu.sync_copy(agg_in.at[1, off], p1buf)
                pltpu.sync_copy(x0_in.at[off], x0buf)

                @plsc.parallel_loop(0, qr // 16, unroll=7)
                def _epi(g):
                    rows = g * 16 + lanes
                    xs = [plsc.load_gather(x0buf, [rows, cols[i]])
                          for i in range(in_cols)]
                    if mean_flag:
                        cnt = (plsc.load_gather(p0buf, [rows, cols[6]])
                               + plsc.load_gather(p1buf, [rows, cols[6]]))
                        den = jnp.maximum(cnt, 1.0)
                        inv = 1.0 / den
                    for k in range(out_cols):
                        a = (plsc.load_gather(p0buf, [rows, cols[k]])
                             + plsc.load_gather(p1buf, [rows, cols[k]]))
                        if mean_flag:
                            a = a * inv
                        for i in range(in_cols):
                            a = a + xs[i] * rsc[i][k]
                        a = jnp.maximum(a + bsc[k], 0.0)
                        plsc.store_scatter(p0buf, [rows, cols[k]], a)
                    if mean_flag:
                        plsc.store_scatter(p0buf, [rows, cols[6]], den)

                pltpu.sync_copy(p0buf, x_sh.at[off])

                @pl.when(c == 0)
                def _():
                    pltpu.sync_copy(p0buf, h_out.at[off])

        @pl.loop(0, 64)
        def _init(g):
            rows = g * 16 + lanes
            for b in range(2):
                if pairsum:
                    for k in (3, 4, 5, 6, 7):
                        plsc.store_scatter(msg.at[b], [rows, cols[k]], zero16)
                else:
                    plsc.store_scatter(msg.at[b], [rows, cols[6]], one16)
                    plsc.store_scatter(msg.at[b], [rows, cols[7]], zero16)

        plsc.subcore_barrier()

        def load_idx(t, b, sem):
            r0 = base + t * CHUNK_ROWS
            e0 = r0 * 128
            hs = [pltpu.async_copy(ei_hbm.at[0, pl.ds(e0, 1024)],
                                   src_buf.at[b], sem),
                  pltpu.async_copy(et_hbm.at[pl.ds(e0, 1024)],
                                   et_buf.at[b], sem)]
            for j in range(CHUNK_ROWS):
                hs.append(pltpu.async_copy(
                    ei_hbm.at[1, pl.ds(e0 + j * 128, 128)],
                    dst_buf.at[t % 4].at[j], sem))
            return hs

        def wait_idx(b):
            pltpu.make_async_copy(ei_hbm.at[0, pl.ds(0, 1024)],
                                  src_buf.at[b], isems[b]).wait()
            pltpu.make_async_copy(et_hbm.at[pl.ds(0, 1024)],
                                  et_buf.at[b], isems[b]).wait()
            for j in range(CHUNK_ROWS):
                pltpu.make_async_copy(ei_hbm.at[1, pl.ds(0, 128)],
                                      dst_buf.at[0].at[j], isems[b]).wait()

        def fire_gathers(b):
            for j in range(CHUNK_ROWS):
                pltpu.async_copy(
                    x_sh.at[src_buf.at[b].at[pl.ds(j * 128, 128)]],
                    xrow.at[b].at[pl.ds(j * 128, 128)], gsems[b])

        def wait_gathers(b):
            pltpu.make_async_copy(z_hbm.at[pl.ds(0, 1024)],
                                  xrow.at[b], gsems[b]).wait()

        def fire_scatters(t, b):
            for j in range(CHUNK_ROWS):
                pltpu.async_copy(msg.at[b].at[pl.ds(j * 128, 128)],
                                 agg_sh.at[dst_buf.at[t % 4].at[j]],
                                 ssems[b], add=True)

        def wait_scatters(b):
            pltpu.make_async_copy(z_hbm.at[pl.ds(0, 1024)],
                                  msg.at[b], ssems[b]).wait()

        def compute(b, ngroups=64):
            @plsc.parallel_loop(0, ngroups, unroll=4)
            def _group(g):
                rows = g * 16 + lanes
                et_v = et_buf.at[b][pl.ds(g * 16, 16)]
                if pairsum:
                    xs = [plsc.load_gather(xrow.at[b], [rows, cols[cc]])
                          for cc in range(6)]
                    for bb in range(3):
                        w0 = plsc.load_gather(wf_buf,
                                              [et_v, cols[2 * bb]])
                        w1 = plsc.load_gather(wf_buf,
                                              [et_v, cols[2 * bb + 1]])
                        plsc.store_scatter(
                            msg.at[b], [rows, cols[bb]],
                            xs[2 * bb] * w0 + xs[2 * bb + 1] * w1)
                else:
                    xs = [plsc.load_gather(xrow.at[b], [rows, cols[cc]])
                          for cc in range(3)]
                    for k in range(6):
                        wk = plsc.load_gather(wf_buf, [et_v, cols[k]])
                        plsc.store_scatter(msg.at[b], [rows, cols[k]],
                                           xs[k >> 1] * wk)

        for h in load_idx(0, 0, isems[0]):
            h.wait()
        fire_gathers(0)
        load_idx(1, 1, isems[1])

        @pl.loop(0, MAIN_CHUNKS // 2)
        def _step(u):
            for phase in range(2):
                t = u * 2 + phase
                b = phase
                nb = 1 - phase

                @pl.when(t + 1 < MAIN_CHUNKS)
                def _():
                    wait_idx(nb)
                    fire_gathers(nb)

                wait_gathers(b)

                @pl.when(t >= 2)
                def _():
                    wait_scatters(b)

                compute(b)
                fire_scatters(t, b)

                @pl.when(t + 2 < MAIN_CHUNKS)
                def _():
                    load_idx(t + 2, b, isems[b])

        wait_scatters(0)
        wait_scatters(1)

        e0t = (base + 384) * 128

        def tail(rr):
            pltpu.sync_copy(ei_hbm.at[0, pl.ds(e0t, rr * 128)],
                            src_buf.at[0].at[pl.ds(0, rr * 128)])
            pltpu.sync_copy(et_hbm.at[pl.ds(e0t, rr * 128)],
                            et_buf.at[0].at[pl.ds(0, rr * 128)])
            for j in range(rr):
                pltpu.sync_copy(ei_hbm.at[1, pl.ds(e0t + j * 128, 128)],
                                dst_buf.at[0].at[j])
            for j in range(rr):
                pltpu.async_copy(
                    x_sh.at[src_buf.at[0].at[pl.ds(j * 128, 128)]],
                    xrow.at[0].at[pl.ds(j * 128, 128)], gsems[0])
            pltpu.make_async_copy(
                z_hbm.at[pl.ds(0, rr * 128)],
                xrow.at[0].at[pl.ds(0, rr * 128)], gsems[0]).wait()
            compute(0, rr * 8)
            for j in range(rr):
                pltpu.sync_copy(msg.at[0].at[pl.ds(j * 128, 128)],
                                agg_sh.at[dst_buf.at[0].at[j]], add=True)

        @pl.when(wid < 20)
        def _():
            tail(7)

        @pl.when(wid >= 20)
        def _():
            tail(6)

        plsc.subcore_barrier()
        pltpu.sync_copy(agg_sh.at[sl], out_hbm.at[c, sl])

    return sc_agg


_sc_l1 = _make_sc_agg(pairsum=False)
_sc_l2 = _make_sc_agg(pairsum=True, epi=(True, 3, 6))
_sc_l3 = _make_sc_agg(pairsum=False, epi=(False, 6, 3))


def _epi_final(x_pad, agg2, h1_pad, rp, bp):
    nblocks = N_PAD // 3136

    def body(x_ref, a_ref, d_ref, r_ref, b_ref, o_ref, acc_ref):
        i = pl.program_id(0)
        x = x_ref[...]
        a = a_ref[0] + a_ref[1]
        denom = d_ref[:, 6:7]
        core = jnp.dot(x, r_ref[...], preferred_element_type=jnp.float32)
        h = jnp.maximum(core[:, :6] + a[:, :6] / denom + b_ref[0, :6], 0.0)
        row = i * 3136 + jax.lax.broadcasted_iota(jnp.int32, (3136, 1), 0)
        h = jnp.where(row < N, h, 0.0)

        @pl.when(i == 0)
        def _():
            acc_ref[...] = jnp.zeros_like(acc_ref)

        acc_ref[...] += h

        @pl.when(i == nblocks - 1)
        def _():
            pooled = jnp.sum(acc_ref[...], axis=0, keepdims=True) / float(N)
            z = pooled - jnp.max(pooled, axis=1, keepdims=True)
            o_ref[...] = z - jnp.log(jnp.sum(jnp.exp(z), axis=1,
                                             keepdims=True))

    return pl.pallas_call(
        body,
        grid=(nblocks,),
        in_specs=[
            pl.BlockSpec((3136, 8), lambda i: (i, 0)),
            pl.BlockSpec((2, 3136, 8), lambda i: (0, i, 0)),
            pl.BlockSpec((3136, 8), lambda i: (i, 0)),
            pl.BlockSpec((8, 8), lambda i: (0, 0)),
            pl.BlockSpec((1, 8), lambda i: (0, 0)),
        ],
        out_specs=pl.BlockSpec((1, 6), lambda i: (0, 0)),
        out_shape=jax.ShapeDtypeStruct((1, 6), jnp.float32),
        scratch_shapes=[pltpu.VMEM((3136, 6), jnp.float32)],
    )(x_pad, agg2, h1_pad, rp, bp)


def _pad_wf(w):
    return jnp.zeros((R + 1, 8), jnp.float32).at[:R, :6].set(
        w.reshape(R, 6).astype(jnp.float32))


def _pad_root(r):
    return jnp.zeros((8, 8), jnp.float32).at[:r.shape[0], :r.shape[1]].set(r)


def _pad_bias(b):
    return jnp.zeros((1, 8), jnp.float32).at[0, :b.shape[0]].set(b)


def kernel(x, edge_index, batch, edge_type, w1, r1, b1, w2, r2, b2,
           w3, r3, b3):
    del batch
    zeros8 = jnp.zeros((N_PAD, 8), jnp.float32)
    x0 = jnp.zeros((N_PAD, 8), jnp.float32).at[:N, :3].set(x)

    agg1 = _sc_l1(x0, edge_index, edge_type, _pad_wf(w1), zeros8)
    agg2, h1 = _sc_l2(edge_index, edge_type, _pad_wf(w2), zeros8,
                      agg1, x0, _pad_root(r1).reshape(64),
                      jnp.zeros((8,), jnp.float32).at[:6].set(b1))
    agg3, h2 = _sc_l3(edge_index, edge_type, _pad_wf(w3), zeros8,
                      agg2, h1, _pad_root(r2).reshape(64),
                      jnp.zeros((8,), jnp.float32).at[:3].set(b2))
    return _epi_final(h2, agg3, h1, _pad_root(r3), _pad_bias(b3))


# speedup vs baseline: 2.5202x; 1.1413x over previous
import dataclasses
import functools

import jax
import jax.numpy as jnp
from jax.experimental import pallas as pl
from jax.experimental.pallas import tpu as pltpu
from jax.experimental.pallas import tpu_sc as plsc

N = 50000
E = 1600000
R = 90

N_PAD = 50176
E_ROWS = E // 128
CHUNK_ROWS = 8
MAIN_CHUNKS = 48
SLICE = N_PAD // 16


def _make_sc_agg(pairsum: bool, epi=None):
    mesh = plsc.VectorSubcoreMesh(core_axis_name="c", subcore_axis_name="s")
    cp = pltpu.CompilerParams()
    for f, v in (("needs_layout_passes", False),
                 ("use_tc_tiling_on_sc", False)):
        if f in pltpu.CompilerParams.__dataclass_fields__:
            cp = dataclasses.replace(cp, **{f: v})

    agg_sds = jax.ShapeDtypeStruct((2, N_PAD, 8), jnp.float32)
    h_sds = jax.ShapeDtypeStruct((N_PAD, 8), jnp.float32)
    scratch = [
        pltpu.VMEM((2, 1024), jnp.int32),
        pltpu.VMEM((2, 1024), jnp.int32),
        pltpu.VMEM((4, CHUNK_ROWS, 128), jnp.int32),
        pltpu.VMEM((2, 1024, 8), jnp.float32),
        pltpu.VMEM((2, 1024, 8), jnp.float32),
        pltpu.VMEM((R + 1, 8), jnp.float32),
        pltpu.VMEM_SHARED((N_PAD, 8), jnp.float32),
        pltpu.VMEM_SHARED((N_PAD, 8), jnp.float32),
        pltpu.SemaphoreType.DMA,
        pltpu.SemaphoreType.DMA,
        pltpu.SemaphoreType.DMA,
        pltpu.SemaphoreType.DMA,
        pltpu.SemaphoreType.DMA,
        pltpu.SemaphoreType.DMA,
    ]
    if epi is not None:
        scratch += [
            pltpu.VMEM((SLICE // 4, 8), jnp.float32),
            pltpu.VMEM((SLICE // 4, 8), jnp.float32),
            pltpu.VMEM((SLICE // 4, 8), jnp.float32),
            pltpu.VMEM((64,), jnp.float32),
            pltpu.VMEM((16,), jnp.float32),
        ]

    @functools.partial(
        pl.kernel,
        compiler_params=cp,
        out_type=agg_sds if epi is None else (agg_sds, h_sds),
        mesh=mesh,
        scratch_types=scratch,
    )
    def sc_agg(*refs):
        if epi is None:
            (x_hbm, ei_hbm, et_hbm, wf_hbm, z_hbm, out_hbm,
             src_buf, et_buf, dst_buf, xrow, msg, wf_buf, agg_sh, x_sh,
             isem0, isem1, gsem0, gsem1, ssem0, ssem1) = refs
        else:
            (ei_hbm, et_hbm, wf_hbm, z_hbm, agg_in, x0_in,
             r_hbm, b_hbm, out_hbm, h_out,
             src_buf, et_buf, dst_buf, xrow, msg, wf_buf, agg_sh, x_sh,
             isem0, isem1, gsem0, gsem1, ssem0, ssem1,
             p0buf, p1buf, x0buf, rbuf, bbuf) = refs
        isems = (isem0, isem1)
        gsems = (gsem0, gsem1)
        ssems = (ssem0, ssem1)
        c = jax.lax.axis_index("c")
        s = jax.lax.axis_index("s")
        wid = c * 16 + s
        lanes = jax.lax.iota(jnp.int32, 16)
        cols = [jnp.full((16,), k, jnp.int32) for k in range(8)]
        zero16 = jnp.zeros((16,), jnp.float32)
        one16 = jnp.ones((16,), jnp.float32)
        base = wid * 390 + jnp.minimum(wid, 20)
        sl = pl.ds(s * SLICE, SLICE)

        pltpu.sync_copy(wf_hbm, wf_buf)
        pltpu.sync_copy(z_hbm.at[sl], agg_sh.at[sl])

        if epi is None:
            pltpu.sync_copy(x_hbm.at[sl], x_sh.at[sl])
        else:
            mean_flag, in_cols, out_cols = epi
            pltpu.sync_copy(r_hbm, rbuf)
            pltpu.sync_copy(b_hbm, bbuf.at[pl.ds(0, 8)])
            rvecs = [rbuf[pl.ds(o * 16, 16)] for o in range(4)]
            rsc = [[rvecs[(i * 8 + k) // 16][(i * 8 + k) % 16]
                    for k in range(out_cols)] for i in range(in_cols)]
            bvec = bbuf[pl.ds(0, 16)]
            bsc = [bvec[k] for k in range(out_cols)]
            qr = SLICE // 4
            for q in range(4):
                off = pl.ds(s * SLICE + q * qr, qr)
                pltpu.sync_copy(agg_in.at[0, off], p0buf)
                pltpu.sync_copy(agg_in.at[1, off], p1buf)
                pltpu.sync_copy(x0_in.at[off], x0buf)

                @plsc.parallel_loop(0, qr // 16, unroll=7)
                def _epi(g):
                    rows = g * 16 + lanes
                    xs = [plsc.load_gather(x0buf, [rows, cols[i]])
                          for i in range(in_cols)]
                    if mean_flag:
                        cnt = (plsc.load_gather(p0buf, [rows, cols[6]])
                               + plsc.load_gather(p1buf, [rows, cols[6]]))
                        den = jnp.maximum(cnt, 1.0)
                        inv = 1.0 / den
                    for k in range(out_cols):
                        a = (plsc.load_gather(p0buf, [rows, cols[k]])
                             + plsc.load_gather(p1buf, [rows, cols[k]]))
                        if mean_flag:
                            a = a * inv
                        for i in range(in_cols):
                            a = a + xs[i] * rsc[i][k]
                        a = jnp.maximum(a + bsc[k], 0.0)
                        plsc.store_scatter(p0buf, [rows, cols[k]], a)
                    if mean_flag:
                        plsc.store_scatter(p0buf, [rows, cols[6]], den)

                pltpu.sync_copy(p0buf, x_sh.at[off])

                @pl.when(c == 0)
                def _():
                    pltpu.sync_copy(p0buf, h_out.at[off])

        @pl.loop(0, 64)
        def _init(g):
            rows = g * 16 + lanes
            for b in range(2):
                if pairsum:
                    for k in (3, 4, 5, 6, 7):
                        plsc.store_scatter(msg.at[b], [rows, cols[k]], zero16)
                else:
                    plsc.store_scatter(msg.at[b], [rows, cols[6]], one16)
                    plsc.store_scatter(msg.at[b], [rows, cols[7]], zero16)

        plsc.subcore_barrier()

        def load_idx(t, b, sem):
            r0 = base + t * CHUNK_ROWS
            e0 = r0 * 128
            hs = [pltpu.async_copy(ei_hbm.at[0, pl.ds(e0, 1024)],
                                   src_buf.at[b], sem),
                  pltpu.async_copy(et_hbm.at[pl.ds(e0, 1024)],
                                   et_buf.at[b], sem)]
            for j in range(CHUNK_ROWS):
                hs.append(pltpu.async_copy(
                    ei_hbm.at[1, pl.ds(e0 + j * 128, 128)],
                    dst_buf.at[t % 4].at[j], sem))
            return hs

        def wait_idx(b):
            pltpu.make_async_copy(ei_hbm.at[0, pl.ds(0, 1024)],
                                  src_buf.at[b], isems[b]).wait()
            pltpu.make_async_copy(et_hbm.at[pl.ds(0, 1024)],
                                  et_buf.at[b], isems[b]).wait()
            for j in range(CHUNK_ROWS):
                pltpu.make_async_copy(ei_hbm.at[1, pl.ds(0, 128)],
                                      dst_buf.at[0].at[j], isems[b]).wait()

        def fire_gathers(b):
            for j in range(CHUNK_ROWS):
                pltpu.async_copy(
                    x_sh.at[src_buf.at[b].at[pl.ds(j * 128, 128)]],
                    xrow.at[b].at[pl.ds(j * 128, 128)], gsems[b])

        def wait_gathers(b):
            pltpu.make_async_copy(z_hbm.at[pl.ds(0, 1024)],
                                  xrow.at[b], gsems[b]).wait()

        def fire_scatters(t, b):
            for j in range(CHUNK_ROWS):
                pltpu.async_copy(msg.at[b].at[pl.ds(j * 128, 128)],
                                 agg_sh.at[dst_buf.at[t % 4].at[j]],
                                 ssems[b], add=True)

        def wait_scatters(b):
            pltpu.make_async_copy(z_hbm.at[pl.ds(0, 1024)],
                                  msg.at[b], ssems[b]).wait()

        def compute(b, ngroups=64):
            @plsc.parallel_loop(0, ngroups, unroll=4)
            def _group(g):
                rows = g * 16 + lanes
                et_v = et_buf.at[b][pl.ds(g * 16, 16)]
                if pairsum:
                    xs = [plsc.load_gather(xrow.at[b], [rows, cols[cc]])
                          for cc in range(6)]
                    for bb in range(3):
                        w0 = plsc.load_gather(wf_buf,
                                              [et_v, cols[2 * bb]])
                        w1 = plsc.load_gather(wf_buf,
                                              [et_v, cols[2 * bb + 1]])
                        plsc.store_scatter(
                            msg.at[b], [rows, cols[bb]],
                            xs[2 * bb] * w0 + xs[2 * bb + 1] * w1)
                else:
                    xs = [plsc.load_gather(xrow.at[b], [rows, cols[cc]])
                          for cc in range(3)]
                    for k in range(6):
                        wk = plsc.load_gather(wf_buf, [et_v, cols[k]])
                        plsc.store_scatter(msg.at[b], [rows, cols[k]],
                                           xs[k >> 1] * wk)

        for h in load_idx(0, 0, isems[0]):
            h.wait()
        fire_gathers(0)
        load_idx(1, 1, isems[1])

        @pl.loop(0, MAIN_CHUNKS // 2)
        def _step(u):
            for phase in range(2):
                t = u * 2 + phase
                b = phase
                nb = 1 - phase

                @pl.when(t + 1 < MAIN_CHUNKS)
                def _():
                    wait_idx(nb)
                    fire_gathers(nb)

                wait_gathers(b)

                @pl.when(t >= 2)
                def _():
                    wait_scatters(b)

                compute(b)
                fire_scatters(t, b)

                @pl.when(t + 2 < MAIN_CHUNKS)
                def _():
                    load_idx(t + 2, b, isems[b])

        wait_scatters(0)
        wait_scatters(1)

        e0t = (base + 384) * 128

        def tail(rr):
            pltpu.sync_copy(ei_hbm.at[0, pl.ds(e0t, rr * 128)],
                            src_buf.at[0].at[pl.ds(0, rr * 128)])
            pltpu.sync_copy(et_hbm.at[pl.ds(e0t, rr * 128)],
                            et_buf.at[0].at[pl.ds(0, rr * 128)])
            for j in range(rr):
                pltpu.sync_copy(ei_hbm.at[1, pl.ds(e0t + j * 128, 128)],
                                dst_buf.at[0].at[j])
            for j in range(rr):
                pltpu.async_copy(
                    x_sh.at[src_buf.at[0].at[pl.ds(j * 128, 128)]],
                    xrow.at[0].at[pl.ds(j * 128, 128)], gsems[0])
            pltpu.make_async_copy(
                z_hbm.at[pl.ds(0, rr * 128)],
                xrow.at[0].at[pl.ds(0, rr * 128)], gsems[0]).wait()
            compute(0, rr * 8)
            for j in range(rr):
                pltpu.sync_copy(msg.at[0].at[pl.ds(j * 128, 128)],
                                agg_sh.at[dst_buf.at[0].at[j]], add=True)

        @pl.when(wid < 20)
        def _():
            tail(7)

        @pl.when(wid >= 20)
        def _():
            tail(6)

        plsc.subcore_barrier()
        pltpu.sync_copy(agg_sh.at[sl], out_hbm.at[c, sl])

    return sc_agg


_sc_l1 = _make_sc_agg(pairsum=False)
_sc_l2 = _make_sc_agg(pairsum=True, epi=(True, 3, 6))
_sc_l3 = _make_sc_agg(pairsum=False, epi=(False, 6, 3))


def _make_sc_pool():
    mesh = plsc.VectorSubcoreMesh(core_axis_name="c", subcore_axis_name="s")
    cp = pltpu.CompilerParams()
    for f, v in (("needs_layout_passes", False),
                 ("use_tc_tiling_on_sc", False)):
        if f in pltpu.CompilerParams.__dataclass_fields__:
            cp = dataclasses.replace(cp, **{f: v})
    pr = N_PAD // 32

    @functools.partial(
        pl.kernel,
        compiler_params=cp,
        out_type=jax.ShapeDtypeStruct((32, 8, 16), jnp.float32),
        mesh=mesh,
        scratch_types=[
            pltpu.VMEM((pr, 8), jnp.float32),
            pltpu.VMEM((pr, 8), jnp.float32),
            pltpu.VMEM((pr, 8), jnp.float32),
            pltpu.VMEM((pr, 8), jnp.float32),
            pltpu.VMEM((64,), jnp.float32),
            pltpu.VMEM((16,), jnp.float32),
            pltpu.VMEM((8, 16), jnp.float32),
        ],
    )
    def sc_pool(agg_in, h2_in, h1_in, r_hbm, b_hbm, out_hbm,
                p0buf, p1buf, x2buf, d1buf, rbuf, bbuf, sbuf):
        c = jax.lax.axis_index("c")
        s = jax.lax.axis_index("s")
        wid = c * 16 + s
        lanes = jax.lax.iota(jnp.int32, 16)
        cols = [jnp.full((16,), k, jnp.int32) for k in range(8)]
        zero16 = jnp.zeros((16,), jnp.float32)
        sl = pl.ds(wid * pr, pr)
        pltpu.sync_copy(agg_in.at[0, sl], p0buf)
        pltpu.sync_copy(agg_in.at[1, sl], p1buf)
        pltpu.sync_copy(h2_in.at[sl], x2buf)
        pltpu.sync_copy(h1_in.at[sl], d1buf)
        pltpu.sync_copy(r_hbm, rbuf)
        pltpu.sync_copy(b_hbm, bbuf.at[pl.ds(0, 8)])
        rvecs = [rbuf[pl.ds(o * 16, 16)] for o in range(4)]
        rsc = [[rvecs[(i * 8 + k) // 16][(i * 8 + k) % 16]
                for k in range(6)] for i in range(3)]
        bvec = bbuf[pl.ds(0, 16)]
        bsc = [bvec[k] for k in range(6)]
        gbase = wid * pr

        def _pool_body(g, acc):
            rows = g * 16 + lanes
            valid = (gbase + rows) < N
            den = plsc.load_gather(d1buf, [rows, cols[6]])
            inv = 1.0 / den
            xs = [plsc.load_gather(x2buf, [rows, cols[i]]) for i in range(3)]
            out = []
            for k in range(6):
                a = (plsc.load_gather(p0buf, [rows, cols[k]])
                     + plsc.load_gather(p1buf, [rows, cols[k]])) * inv
                for i in range(3):
                    a = a + xs[i] * rsc[i][k]
                a = jnp.maximum(a + bsc[k], 0.0)
                a = jnp.where(valid, a, 0.0)
                out.append(acc[k] + a)
            return tuple(out)

        accs = plsc.parallel_loop(0, pr // 16, unroll=2,
                                  carry=(zero16,) * 6)(_pool_body)
        for k in range(6):
            sbuf.at[k][...] = accs[k]
        sbuf.at[6][...] = zero16
        sbuf.at[7][...] = zero16
        pltpu.sync_copy(sbuf, out_hbm.at[wid])

    return sc_pool


_sc_pool = _make_sc_pool()


def _final_tc(pool):
    def body(p_ref, o_ref):
        tot = jnp.sum(p_ref[...], axis=(0, 2)) / float(N)
        z = tot[:6].reshape(1, 6)
        z = z - jnp.max(z, axis=1, keepdims=True)
        o_ref[...] = z - jnp.log(jnp.sum(jnp.exp(z), axis=1, keepdims=True))

    return pl.pallas_call(
        body,
        out_shape=jax.ShapeDtypeStruct((1, 6), jnp.float32),
    )(pool)


def _epi_final(x_pad, agg2, h1_pad, rp, bp):
    nblocks = N_PAD // 3136

    def body(x_ref, a_ref, d_ref, r_ref, b_ref, o_ref, acc_ref):
        i = pl.program_id(0)
        x = x_ref[...]
        a = a_ref[0] + a_ref[1]
        denom = d_ref[:, 6:7]
        core = jnp.dot(x, r_ref[...], preferred_element_type=jnp.float32)
        h = jnp.maximum(core[:, :6] + a[:, :6] / denom + b_ref[0, :6], 0.0)
        row = i * 3136 + jax.lax.broadcasted_iota(jnp.int32, (3136, 1), 0)
        h = jnp.where(row < N, h, 0.0)

        @pl.when(i == 0)
        def _():
            acc_ref[...] = jnp.zeros_like(acc_ref)

        acc_ref[...] += h

        @pl.when(i == nblocks - 1)
        def _():
            pooled = jnp.sum(acc_ref[...], axis=0, keepdims=True) / float(N)
            z = pooled - jnp.max(pooled, axis=1, keepdims=True)
            o_ref[...] = z - jnp.log(jnp.sum(jnp.exp(z), axis=1,
                                             keepdims=True))

    return pl.pallas_call(
        body,
        grid=(nblocks,),
        in_specs=[
            pl.BlockSpec((3136, 8), lambda i: (i, 0)),
            pl.BlockSpec((2, 3136, 8), lambda i: (0, i, 0)),
            pl.BlockSpec((3136, 8), lambda i: (i, 0)),
            pl.BlockSpec((8, 8), lambda i: (0, 0)),
            pl.BlockSpec((1, 8), lambda i: (0, 0)),
        ],
        out_specs=pl.BlockSpec((1, 6), lambda i: (0, 0)),
        out_shape=jax.ShapeDtypeStruct((1, 6), jnp.float32),
        scratch_shapes=[pltpu.VMEM((3136, 6), jnp.float32)],
    )(x_pad, agg2, h1_pad, rp, bp)


def _pad_wf(w):
    return jnp.zeros((R + 1, 8), jnp.float32).at[:R, :6].set(
        w.reshape(R, 6).astype(jnp.float32))


def _pad_root(r):
    return jnp.zeros((8, 8), jnp.float32).at[:r.shape[0], :r.shape[1]].set(r)


def _pad_bias(b):
    return jnp.zeros((1, 8), jnp.float32).at[0, :b.shape[0]].set(b)


def kernel(x, edge_index, batch, edge_type, w1, r1, b1, w2, r2, b2,
           w3, r3, b3):
    del batch
    zeros8 = jnp.zeros((N_PAD, 8), jnp.float32)
    x0 = jnp.zeros((N_PAD, 8), jnp.float32).at[:N, :3].set(x)

    agg1 = _sc_l1(x0, edge_index, edge_type, _pad_wf(w1), zeros8)
    agg2, h1 = _sc_l2(edge_index, edge_type, _pad_wf(w2), zeros8,
                      agg1, x0, _pad_root(r1).reshape(64),
                      jnp.zeros((8,), jnp.float32).at[:6].set(b1))
    agg3, h2 = _sc_l3(edge_index, edge_type, _pad_wf(w3), zeros8,
                      agg2, h1, _pad_root(r2).reshape(64),
                      jnp.zeros((8,), jnp.float32).at[:3].set(b2))
    pool = _sc_pool(agg3, h2, h1, _pad_root(r3).reshape(64),
                    jnp.zeros((8,), jnp.float32).at[:6].set(b3))
    return _final_tc(pool)
